# baseline (speedup 1.0000x reference)
import functools

import jax
import jax.numpy as jnp
from jax.experimental import pallas as pl


def _leaky(x):
    return jnp.where(x > 0, x, 0.2 * x)


def _mm_kernel(x_ref, w_ref, o_ref, *, act):
    acc = jax.lax.dot_general(
        x_ref[...], w_ref[...],
        dimension_numbers=(((1,), (0,)), ((), ())),
        preferred_element_type=jnp.float32,
    )
    if act:
        acc = _leaky(acc)
    o_ref[...] = acc


def _pad_to(a, mult0, mult1):
    m0 = (-a.shape[0]) % mult0
    m1 = (-a.shape[1]) % mult1
    if m0 or m1:
        a = jnp.pad(a, ((0, m0), (0, m1)))
    return a


@functools.partial(jax.jit, static_argnames=("act",))
def _mm(x, w, act=False):
    M, K = x.shape
    K2, N = w.shape
    assert K == K2
    BM, BN = 256, 256
    xp = _pad_to(x, BM, 128)
    wp = _pad_to(w, 128, BN)
    Mp, Kp = xp.shape
    Np = wp.shape[1]
    out = pl.pallas_call(
        functools.partial(_mm_kernel, act=act),
        grid=(Mp // BM, Np // BN),
        in_specs=[
            pl.BlockSpec((BM, Kp), lambda i, j: (i, 0)),
            pl.BlockSpec((Kp, BN), lambda i, j: (0, j)),
        ],
        out_specs=pl.BlockSpec((BM, BN), lambda i, j: (i, j)),
        out_shape=jax.ShapeDtypeStruct((Mp, Np), jnp.float32),
    )(xp, wp)
    return out[:M, :N]


def _segment_mean(data, seg, num):
    s = jax.ops.segment_sum(data, seg, num_segments=num)
    c = jax.ops.segment_sum(jnp.ones((data.shape[0],), data.dtype), seg,
                            num_segments=num)
    return s / jnp.clip(c, 1.0)[:, None]


def kernel(x_win, x_edge, edge_index_near, edge_index_close, edge_index_sim,
           ij2idx_near, ij2idx_close, ij2idx_sim, edge_edge_index,
           W_pre_win, W_post_win, W_pre_edge, W_post_edge,
           W_rel_self, W_rel_nbr, pool_q, pool_W,
           edge_Wl, edge_Wr, edge_b, W_out, b_out):
    H = W_pre_win.shape[1]
    L = W_rel_self.shape[0]
    n = x_win.shape[0]
    m = x_edge.shape[0]

    xw = _mm(_mm(x_win, W_pre_win, act=True), W_post_win, act=True)
    xe = _mm(_mm(x_edge, W_pre_edge, act=True), W_post_edge, act=True)

    rels = [(edge_index_near, ij2idx_near),
            (edge_index_close, ij2idx_close),
            (edge_index_sim, ij2idx_sim)]
    es, ed = edge_edge_index[0], edge_edge_index[1]

    for l in range(L):
        att = jax.nn.softmax(xw @ pool_q[l] / jnp.sqrt(jnp.float32(H)))
        g = (att @ xw) @ pool_W[l]
        acc = _mm(xw, W_rel_self[l].sum(axis=0))
        for r, (ei, ij) in enumerate(rels):
            src, dst = ei[0], ei[1]
            msg = xw[src] + xe[ij] + g[None, :]
            acc = acc + _mm(_segment_mean(msg, dst, n), W_rel_nbr[l, r])
        xw_new = acc / 3.0

        aggr_e = _segment_mean(xe[es], ed, m)
        xe = _leaky(_mm(aggr_e, edge_Wl[l]) + _mm(xe, edge_Wr[l]) + edge_b[l])
        xw = _leaky(xw_new)

    return _mm(xw, W_out) + b_out


# speedup vs baseline: 1.0305x; 1.0305x over previous
import functools

import jax
import jax.numpy as jnp
from jax.experimental import pallas as pl


def _leaky(x):
    return jnp.where(x > 0, x, 0.2 * x)


def _mm_kernel(x_ref, w_ref, o_ref, *, act):
    acc = jax.lax.dot_general(
        x_ref[...], w_ref[...],
        dimension_numbers=(((1,), (0,)), ((), ())),
        preferred_element_type=jnp.float32,
    )
    if act:
        acc = _leaky(acc)
    o_ref[...] = acc


def _pad_to(a, mult0, mult1):
    m0 = (-a.shape[0]) % mult0
    m1 = (-a.shape[1]) % mult1
    if m0 or m1:
        a = jnp.pad(a, ((0, m0), (0, m1)))
    return a


@functools.partial(jax.jit, static_argnames=("act",))
def _mm(x, w, act=False):
    M, K = x.shape
    K2, N = w.shape
    assert K == K2
    BM = 512
    BN = -(-N // 128) * 128
    xp = _pad_to(x, BM, 128)
    wp = _pad_to(w, 128, BN)
    Mp, Kp = xp.shape
    Np = wp.shape[1]
    out = pl.pallas_call(
        functools.partial(_mm_kernel, act=act),
        grid=(Mp // BM, Np // BN),
        in_specs=[
            pl.BlockSpec((BM, Kp), lambda i, j: (i, 0)),
            pl.BlockSpec((Kp, BN), lambda i, j: (0, j)),
        ],
        out_specs=pl.BlockSpec((BM, BN), lambda i, j: (i, j)),
        out_shape=jax.ShapeDtypeStruct((Mp, Np), jnp.float32),
    )(xp, wp)
    return out[:M, :N]


def _segment_mean(data, seg, num):
    s = jax.ops.segment_sum(data, seg, num_segments=num)
    c = jax.ops.segment_sum(jnp.ones((data.shape[0],), data.dtype), seg,
                            num_segments=num)
    return s / jnp.clip(c, 1.0)[:, None]


def kernel(x_win, x_edge, edge_index_near, edge_index_close, edge_index_sim,
           ij2idx_near, ij2idx_close, ij2idx_sim, edge_edge_index,
           W_pre_win, W_post_win, W_pre_edge, W_post_edge,
           W_rel_self, W_rel_nbr, pool_q, pool_W,
           edge_Wl, edge_Wr, edge_b, W_out, b_out):
    H = W_pre_win.shape[1]
    L = W_rel_self.shape[0]
    n = x_win.shape[0]
    m = x_edge.shape[0]

    xw = _mm(_mm(x_win, W_pre_win, act=True), W_post_win, act=True)
    xe = _mm(_mm(x_edge, W_pre_edge, act=True), W_post_edge, act=True)

    rels = [(edge_index_near, ij2idx_near),
            (edge_index_close, ij2idx_close),
            (edge_index_sim, ij2idx_sim)]
    es, ed = edge_edge_index[0], edge_edge_index[1]

    for l in range(L):
        att = jax.nn.softmax(xw @ pool_q[l] / jnp.sqrt(jnp.float32(H)))
        g = (att @ xw) @ pool_W[l]
        acc = _mm(xw, W_rel_self[l].sum(axis=0) / 3.0)
        for r, (ei, ij) in enumerate(rels):
            src, dst = ei[0], ei[1]
            msg = xw[src] + xe[ij] + g[None, :]
            acc = acc + _mm(_segment_mean(msg, dst, n), W_rel_nbr[l, r] / 3.0)
        xw_new = acc

        aggr_e = _segment_mean(xe[es], ed, m)
        xe = _leaky(_mm(aggr_e, edge_Wl[l]) + _mm(xe, edge_Wr[l]) + edge_b[l])
        xw = _leaky(xw_new)

    return _mm(xw, W_out) + b_out


# speedup vs baseline: 1.1631x; 1.1287x over previous
import functools

import jax
import jax.numpy as jnp
from jax.experimental import pallas as pl


def _leaky(x):
    return jnp.where(x > 0, x, 0.2 * x)


def _mm_kernel(x_ref, w_ref, o_ref, *, act):
    acc = jax.lax.dot_general(
        x_ref[...], w_ref[...],
        dimension_numbers=(((1,), (0,)), ((), ())),
        preferred_element_type=jnp.float32,
    )
    if act:
        acc = _leaky(acc)
    o_ref[...] = acc


def _pad_to(a, mult0, mult1):
    m0 = (-a.shape[0]) % mult0
    m1 = (-a.shape[1]) % mult1
    if m0 or m1:
        a = jnp.pad(a, ((0, m0), (0, m1)))
    return a


@functools.partial(jax.jit, static_argnames=("act",))
def _mm(x, w, act=False):
    M, K = x.shape
    K2, N = w.shape
    assert K == K2
    BM = 512
    BN = -(-N // 128) * 128
    xp = _pad_to(x, BM, 128)
    wp = _pad_to(w, 128, BN)
    Mp, Kp = xp.shape
    Np = wp.shape[1]
    out = pl.pallas_call(
        functools.partial(_mm_kernel, act=act),
        grid=(Mp // BM, Np // BN),
        in_specs=[
            pl.BlockSpec((BM, Kp), lambda i, j: (i, 0)),
            pl.BlockSpec((Kp, BN), lambda i, j: (0, j)),
        ],
        out_specs=pl.BlockSpec((BM, BN), lambda i, j: (i, j)),
        out_shape=jax.ShapeDtypeStruct((Mp, Np), jnp.float32),
    )(xp, wp)
    return out[:M, :N]


def _segment_mean(data, seg, num):
    s = jax.ops.segment_sum(data, seg, num_segments=num)
    c = jax.ops.segment_sum(jnp.ones((data.shape[0],), data.dtype), seg,
                            num_segments=num)
    return s / jnp.clip(c, 1.0)[:, None]


def kernel(x_win, x_edge, edge_index_near, edge_index_close, edge_index_sim,
           ij2idx_near, ij2idx_close, ij2idx_sim, edge_edge_index,
           W_pre_win, W_post_win, W_pre_edge, W_post_edge,
           W_rel_self, W_rel_nbr, pool_q, pool_W,
           edge_Wl, edge_Wr, edge_b, W_out, b_out):
    H = W_pre_win.shape[1]
    L = W_rel_self.shape[0]
    n = x_win.shape[0]
    m = x_edge.shape[0]

    xw = _mm(_mm(x_win, W_pre_win, act=True), W_post_win, act=True)
    xe = _mm(_mm(x_edge, W_pre_edge, act=True), W_post_edge, act=True)

    E = edge_index_near.shape[1]
    srcs = jnp.concatenate([edge_index_near[0], edge_index_close[0],
                            edge_index_sim[0]])
    dsts = jnp.concatenate([edge_index_near[1], edge_index_close[1] + n,
                            edge_index_sim[1] + 2 * n])
    ijs = jnp.concatenate([ij2idx_near, ij2idx_close, ij2idx_sim])
    es, ed = edge_edge_index[0], edge_edge_index[1]

    for l in range(L):
        att = jax.nn.softmax(xw @ pool_q[l] / jnp.sqrt(jnp.float32(H)))
        g = (att @ xw) @ pool_W[l]
        acc = _mm(xw, W_rel_self[l].sum(axis=0) / 3.0)
        msg = xw[srcs] + xe[ijs] + g[None, :]
        aggr = _segment_mean(msg, dsts, 3 * n)
        for r in range(3):
            acc = acc + _mm(aggr[r * n:(r + 1) * n], W_rel_nbr[l, r] / 3.0)
        xw_new = acc

        aggr_e = _segment_mean(xe[es], ed, m)
        xe = _leaky(_mm(aggr_e, edge_Wl[l]) + _mm(xe, edge_Wr[l]) + edge_b[l])
        xw = _leaky(xw_new)

    return _mm(xw, W_out) + b_out
